# baseline (speedup 1.0000x reference)
import functools

import jax
import jax.numpy as jnp
from jax import lax
from jax.experimental import pallas as pl
from jax.experimental.pallas import tpu as pltpu
from jax.experimental.pallas import tpu_sc as plsc

B = 16384
F = 26
V = 100000
D = 13
NC = 2
NS = 16
L = 16
FPC = F // NC
CHUNK = B // 2
B_CMB = B // (NC * NS)


@functools.lru_cache(maxsize=1)
def _build():
    mesh = plsc.VectorSubcoreMesh(core_axis_name="c", subcore_axis_name="s",
                                  num_cores=NC, num_subcores=NS)
    params = pltpu.CompilerParams(needs_layout_passes=False)

    @functools.partial(
        pl.kernel,
        out_type=jax.ShapeDtypeStruct((F * B,), jnp.float32),
        mesh=mesh,
        compiler_params=params,
        scratch_types=[
            pltpu.VMEM((V,), jnp.float32),
            pltpu.VMEM((CHUNK,), jnp.int32),
            pltpu.VMEM((B,), jnp.float32),
        ],
    )
    def _lookup(w_hbm, idx_hbm, out_hbm, row_v, idx_v, col_v):
        c = lax.axis_index("c")
        s = lax.axis_index("s")

        @pl.when(s < FPC)
        def _gather_field():
            f = c * FPC + s
            pltpu.sync_copy(w_hbm.at[pl.ds(f * V, V)], row_v)
            for h in range(B // CHUNK):
                pltpu.sync_copy(
                    idx_hbm.at[pl.ds(f * B + h * CHUNK, CHUNK)], idx_v)

                @plsc.parallel_loop(0, CHUNK, L, unroll=8)
                def _(i):
                    col_v[pl.ds(h * CHUNK + i, L)] = plsc.load_gather(
                        row_v, [idx_v[pl.ds(i, L)]])

            pltpu.sync_copy(col_v, out_hbm.at[pl.ds(f * B, B)])

    @functools.partial(
        pl.kernel,
        out_type=jax.ShapeDtypeStruct((B,), jnp.float32),
        mesh=mesh,
        compiler_params=params,
        scratch_types=[
            pltpu.VMEM((F, B_CMB), jnp.float32),
            pltpu.VMEM((D, B_CMB), jnp.float32),
            pltpu.VMEM((D, L), jnp.float32),
            pltpu.VMEM((L,), jnp.float32),
            pltpu.VMEM((B_CMB,), jnp.float32),
        ],
    )
    def _combine(cols_hbm, dense_hbm, wd_hbm, c0_hbm, out_hbm, g_v, dd_v,
                 wd_v, c0_v, o_v):
        c = lax.axis_index("c")
        s = lax.axis_index("s")
        base = (s * NC + c) * B_CMB
        pltpu.sync_copy(cols_hbm.at[:, pl.ds(base, B_CMB)], g_v)
        pltpu.sync_copy(dense_hbm.at[:, pl.ds(base, B_CMB)], dd_v)
        pltpu.sync_copy(wd_hbm, wd_v)
        pltpu.sync_copy(c0_hbm, c0_v)

        @plsc.parallel_loop(0, B_CMB, L, unroll=2)
        def _(i):
            acc = c0_v[...]
            for f in range(F):
                acc = acc + g_v[f, pl.ds(i, L)]
            for d in range(D):
                acc = acc + wd_v[d] * dd_v[d, pl.ds(i, L)]
            o_v[pl.ds(i, L)] = 1.0 / (1.0 + jnp.exp(-acc))

        pltpu.sync_copy(o_v, out_hbm.at[pl.ds(base, B_CMB)])

    return _lookup, _combine


def kernel(sparse_features, dense_features, W_sparse, W_dense, b_dense, bias):
    w_flat = W_sparse.reshape(F * V)
    idx_flat = sparse_features.T.reshape(F * B)
    dense_t = dense_features.T
    wd_b = jnp.broadcast_to(W_dense.reshape(D, 1), (D, L))
    c0 = jnp.broadcast_to((bias + b_dense).reshape(1), (L,))
    lookup, combine = _build()
    cols = lookup(w_flat, idx_flat).reshape(F, B)
    return combine(cols, dense_t, wd_b, c0)


# speedup vs baseline: 1.1144x; 1.1144x over previous
import functools

import jax
import jax.numpy as jnp
from jax import lax
from jax.experimental import pallas as pl
from jax.experimental.pallas import tpu as pltpu
from jax.experimental.pallas import tpu_sc as plsc

B = 16384
F = 26
V = 100000
D = 13
NC = 2
NS = 16
L = 16
FPC = F // NC
CHUNK = B // 2


@functools.lru_cache(maxsize=1)
def _build():
    mesh = plsc.VectorSubcoreMesh(core_axis_name="c", subcore_axis_name="s",
                                  num_cores=NC, num_subcores=NS)
    params = pltpu.CompilerParams(needs_layout_passes=False)

    @functools.partial(
        pl.kernel,
        out_type=jax.ShapeDtypeStruct((F * B,), jnp.float32),
        mesh=mesh,
        compiler_params=params,
        scratch_types=[
            pltpu.VMEM((V,), jnp.float32),
            pltpu.VMEM((CHUNK,), jnp.int32),
            pltpu.VMEM((B,), jnp.float32),
        ],
    )
    def _lookup(w_hbm, idx_hbm, out_hbm, row_v, idx_v, col_v):
        c = lax.axis_index("c")
        s = lax.axis_index("s")

        @pl.when(s < FPC)
        def _gather_field():
            f = c * FPC + s
            pltpu.sync_copy(w_hbm.at[pl.ds(f * V, V)], row_v)
            for h in range(B // CHUNK):
                pltpu.sync_copy(
                    idx_hbm.at[pl.ds(f * B + h * CHUNK, CHUNK)], idx_v)

                @plsc.parallel_loop(0, CHUNK, L, unroll=8)
                def _(i):
                    col_v[pl.ds(h * CHUNK + i, L)] = plsc.load_gather(
                        row_v, [idx_v[pl.ds(i, L)]])

            pltpu.sync_copy(col_v, out_hbm.at[pl.ds(f * B, B)])

    return _lookup


def _combine_tc(cols_ref, dense_ref, wd_ref, c0_ref, out_ref):
    s = jnp.sum(cols_ref[...], axis=0, keepdims=True)
    dn = jnp.sum(dense_ref[...] * wd_ref[...], axis=0, keepdims=True)
    x = s + dn + c0_ref[0, 0]
    out_ref[...] = 1.0 / (1.0 + jnp.exp(-x))


def kernel(sparse_features, dense_features, W_sparse, W_dense, b_dense, bias):
    w_flat = W_sparse.reshape(F * V)
    idx_flat = sparse_features.T.reshape(F * B)
    dense_t = dense_features.T
    wd = W_dense.reshape(D, 1)
    c0 = (bias + b_dense).reshape(1, 1)
    lookup = _build()
    cols = lookup(w_flat, idx_flat).reshape(F, B)
    out = pl.pallas_call(
        _combine_tc,
        out_shape=jax.ShapeDtypeStruct((1, B), jnp.float32),
    )(cols, dense_t, wd, c0)
    return out.reshape(B)


# speedup vs baseline: 1.7336x; 1.5556x over previous
import functools

import jax
import jax.numpy as jnp
from jax import lax
from jax.experimental import pallas as pl
from jax.experimental.pallas import tpu as pltpu
from jax.experimental.pallas import tpu_sc as plsc

B = 16384
F = 26
V = 100000
D = 13
NC = 2
NS = 16
L = 16
FPC = F // NC
CHUNK = B // 2


@functools.lru_cache(maxsize=1)
def _build():
    mesh = plsc.VectorSubcoreMesh(core_axis_name="c", subcore_axis_name="s",
                                  num_cores=NC, num_subcores=NS)
    params = pltpu.CompilerParams(needs_layout_passes=False)

    @functools.partial(
        pl.kernel,
        out_type=jax.ShapeDtypeStruct((F, B), jnp.float32),
        mesh=mesh,
        compiler_params=params,
        scratch_types=[
            pltpu.VMEM((1, V), jnp.float32),
            pltpu.VMEM((1, CHUNK), jnp.int32),
            pltpu.VMEM((1, B), jnp.float32),
        ],
    )
    def _lookup(w_hbm, idx_hbm, out_hbm, row_v, idx_v, col_v):
        c = lax.axis_index("c")
        s = lax.axis_index("s")

        @pl.when(s < FPC)
        def _gather_field():
            f = c * FPC + s
            zero = jnp.zeros((L,), jnp.int32)
            pltpu.sync_copy(w_hbm.at[pl.ds(f, 1), :], row_v)
            for h in range(B // CHUNK):
                pltpu.sync_copy(
                    idx_hbm.at[pl.ds(f, 1), pl.ds(h * CHUNK, CHUNK)], idx_v)

                @plsc.parallel_loop(0, CHUNK, L, unroll=8)
                def _(i):
                    col_v[0, pl.ds(h * CHUNK + i, L)] = plsc.load_gather(
                        row_v, [zero, idx_v[0, pl.ds(i, L)]])

            pltpu.sync_copy(col_v, out_hbm.at[pl.ds(f, 1), :])

    return _lookup


def _combine_tc(cols_ref, dense_ref, wd_ref, c0_ref, out_ref):
    s = jnp.sum(cols_ref[...], axis=0, keepdims=True)
    dn = jnp.sum(dense_ref[...] * wd_ref[...], axis=0, keepdims=True)
    x = s + dn + c0_ref[0, 0]
    out_ref[...] = 1.0 / (1.0 + jnp.exp(-x))


def kernel(sparse_features, dense_features, W_sparse, W_dense, b_dense, bias):
    idx_t = sparse_features.T
    dense_t = dense_features.T
    wd = W_dense.reshape(D, 1)
    c0 = (bias + b_dense).reshape(1, 1)
    lookup = _build()
    cols = lookup(W_sparse, idx_t)
    out = pl.pallas_call(
        _combine_tc,
        out_shape=jax.ShapeDtypeStruct((1, B), jnp.float32),
    )(cols, dense_t, wd, c0)
    return out.reshape(B)
